# baseline (speedup 1.0000x reference)
import functools

import jax
import jax.numpy as jnp
from jax.experimental import pallas as pl
from jax.experimental.pallas import tpu as pltpu

N = 2048


def _cog_kernel(x_ref, adj_ref, W1_ref, b1_ref, W2_ref, b2_ref, out_ref):
    dot = functools.partial(
        jax.lax.dot_general,
        precision=jax.lax.Precision.HIGHEST,
        preferred_element_type=jnp.float32,
    )

    adj = adj_ref[...]
    deg = jnp.sum(adj, axis=0, keepdims=True) + 1.0
    pos = deg > 0.0
    dinv = jnp.where(pos, jax.lax.rsqrt(jnp.where(pos, deg, 1.0)), 0.0)
    dinv2 = dinv * dinv

    def aggregate(zt, b_ref):
        aggt = jax.lax.dot_general(
            dinv * zt, adj, (((1,), (0,)), ((), ())),
            precision=jax.lax.Precision.DEFAULT,
            preferred_element_type=jnp.float32,
        )
        return dinv * aggt + dinv2 * zt + b_ref[...]

    zt0 = dot(W1_ref[...], x_ref[...], (((1,), (1,)), ((), ())))
    h1t = jnp.maximum(aggregate(zt0, b1_ref), 0.0)
    zt1 = dot(W2_ref[...], h1t, (((1,), (0,)), ((), ())))
    logits = aggregate(zt1, b2_ref) * 5.0

    m = jnp.max(logits, axis=0, keepdims=True)
    s = logits - m
    lse = jnp.log(jnp.sum(jnp.exp(s), axis=0, keepdims=True))
    out_ref[...] = jnp.transpose(s - lse)


def kernel(x, adj, W1, b1, W2, b2):
    nclass = W2.shape[0]
    return pl.pallas_call(
        _cog_kernel,
        out_shape=jax.ShapeDtypeStruct((N, nclass), jnp.float32),
        compiler_params=pltpu.CompilerParams(
            vmem_limit_bytes=100 * 1024 * 1024,
        ),
    )(x, adj, W1, b1[:, None], W2, b2[:, None])


# speedup vs baseline: 1.4389x; 1.4389x over previous
import functools

import jax
import jax.numpy as jnp
from jax.experimental import pallas as pl
from jax.experimental.pallas import tpu as pltpu

N = 2048
CHUNKS = 4
ROWS = N // CHUNKS


def _cog_kernel(xt_ref, adj_hbm, W1_ref, b1_ref, W2_ref, b2_ref, out_ref,
                adj_vmem, sems):
    dot = functools.partial(
        jax.lax.dot_general,
        precision=jax.lax.Precision.DEFAULT,
        preferred_element_type=jnp.float32,
    )

    copies = []
    for k in range(CHUNKS):
        cp = pltpu.make_async_copy(
            adj_hbm.at[pl.ds(k * ROWS, ROWS), :],
            adj_vmem.at[pl.ds(k * ROWS, ROWS), :],
            sems.at[k],
        )
        cp.start()
        copies.append(cp)

    zt0 = dot(W1_ref[...], xt_ref[...], (((1,), (0,)), ((), ())))

    deg = jnp.ones((1, N), dtype=jnp.float32)
    for k in range(CHUNKS):
        copies[k].wait()
        deg = deg + jnp.sum(adj_vmem[pl.ds(k * ROWS, ROWS), :], axis=0,
                            keepdims=True)
    pos = deg > 0.0
    dinv = jnp.where(pos, jax.lax.rsqrt(jnp.where(pos, deg, 1.0)), 0.0)
    dinv2 = dinv * dinv

    adj = adj_vmem[...]

    def aggregate(zt, b_ref):
        aggt = dot(dinv * zt, adj, (((1,), (0,)), ((), ())))
        return dinv * aggt + dinv2 * zt + b_ref[...]

    h1t = jnp.maximum(aggregate(zt0, b1_ref), 0.0)
    zt1 = dot(W2_ref[...], h1t, (((1,), (0,)), ((), ())))
    logits = aggregate(zt1, b2_ref) * 5.0

    m = jnp.max(logits, axis=0, keepdims=True)
    s = logits - m
    lse = jnp.log(jnp.sum(jnp.exp(s), axis=0, keepdims=True))
    out_ref[...] = s - lse


def kernel(x, adj, W1, b1, W2, b2):
    nclass = W2.shape[0]
    out_t = pl.pallas_call(
        _cog_kernel,
        out_shape=jax.ShapeDtypeStruct((nclass, N), jnp.float32),
        in_specs=[
            pl.BlockSpec(memory_space=pltpu.MemorySpace.VMEM),
            pl.BlockSpec(memory_space=pl.ANY),
            pl.BlockSpec(memory_space=pltpu.MemorySpace.VMEM),
            pl.BlockSpec(memory_space=pltpu.MemorySpace.VMEM),
            pl.BlockSpec(memory_space=pltpu.MemorySpace.VMEM),
            pl.BlockSpec(memory_space=pltpu.MemorySpace.VMEM),
        ],
        scratch_shapes=[
            pltpu.VMEM((N, N), jnp.float32),
            pltpu.SemaphoreType.DMA((CHUNKS,)),
        ],
        compiler_params=pltpu.CompilerParams(
            vmem_limit_bytes=100 * 1024 * 1024,
        ),
    )(x.T, adj, W1, b1[:, None], W2, b2[:, None])
    return out_t.T


# speedup vs baseline: 1.4728x; 1.0236x over previous
import functools

import jax
import jax.numpy as jnp
from jax.experimental import pallas as pl
from jax.experimental.pallas import tpu as pltpu

N = 2048


def _cog_kernel(xt_ref, adj_ref, W1_ref, b1_ref, W2_ref, b2_ref, out_ref):
    dot = functools.partial(
        jax.lax.dot_general,
        precision=jax.lax.Precision.DEFAULT,
        preferred_element_type=jnp.float32,
    )

    adj = adj_ref[...]
    deg = jnp.sum(adj, axis=0, keepdims=True) + 1.0
    pos = deg > 0.0
    dinv = jnp.where(pos, jax.lax.rsqrt(jnp.where(pos, deg, 1.0)), 0.0)
    dinv2 = dinv * dinv

    def gcn_layer(ht, W_ref, b_ref):
        zt = dot(W_ref[...], ht, (((1,), (0,)), ((), ())))
        aggt = dot(dinv * zt, adj, (((1,), (0,)), ((), ())))
        return dinv * aggt + dinv2 * zt + b_ref[...]

    h1t = jnp.maximum(gcn_layer(xt_ref[...], W1_ref, b1_ref), 0.0)
    logits = gcn_layer(h1t, W2_ref, b2_ref) * 5.0

    m = jnp.max(logits, axis=0, keepdims=True)
    s = logits - m
    lse = jnp.log(jnp.sum(jnp.exp(s), axis=0, keepdims=True))
    out_ref[...] = s - lse


def kernel(x, adj, W1, b1, W2, b2):
    nclass = W2.shape[0]
    out_t = pl.pallas_call(
        _cog_kernel,
        out_shape=jax.ShapeDtypeStruct((nclass, N), jnp.float32),
        compiler_params=pltpu.CompilerParams(
            vmem_limit_bytes=100 * 1024 * 1024,
        ),
    )(x.T, adj, W1, b1[:, None], W2, b2[:, None])
    return out_t.T
